# Initial kernel scaffold; baseline (speedup 1.0000x reference)
import functools

import jax
import jax.numpy as jnp
from jax import lax
from jax.experimental import pallas as pl
from jax.experimental.pallas import tpu as pltpu
from jax.experimental.pallas import tpu_sc as plsc

B = 1024
L = 20
D = 128
V = 1000

N_IDX = B * L

_INFO = plsc.get_sparse_core_info()
NW = _INFO.num_cores * _INFO.num_subcores
ROWS_PER_W = N_IDX // NW
CHUNK = 128
N_CHUNKS = ROWS_PER_W // CHUNK
TOTAL_CHUNKS = N_IDX // CHUNK


def _gather_body(emb_hbm, idx_hbm, out_hbm, idx_v, rows_v, sem):
    wid = lax.axis_index("s") * _INFO.num_cores + lax.axis_index("c")
    base = wid * N_CHUNKS
    pltpu.sync_copy(idx_hbm.at[pl.ds(base, N_CHUNKS)], idx_v)
    copies = []
    for j in range(N_CHUNKS):
        copies.append(
            pltpu.async_copy(emb_hbm.at[idx_v.at[j]], rows_v.at[j], sem)
        )
    for c in copies:
        c.wait()
    pltpu.sync_copy(rows_v, out_hbm.at[pl.ds(base, N_CHUNKS)])


@functools.partial(
    pl.kernel,
    out_type=jax.ShapeDtypeStruct((TOTAL_CHUNKS, CHUNK, D), jnp.float32),
    mesh=plsc.VectorSubcoreMesh(core_axis_name="c", subcore_axis_name="s"),
    scratch_types=[
        pltpu.VMEM((N_CHUNKS, CHUNK), jnp.int32),
        pltpu.VMEM((N_CHUNKS, CHUNK, D), jnp.float32),
        pltpu.SemaphoreType.DMA,
    ],
)
def _sc_gather(emb_hbm, idx_hbm, out_hbm, idx_v, rows_v, sem):
    _gather_body(emb_hbm, idx_hbm, out_hbm, idx_v, rows_v, sem)


BS = 256


def _matmul_body(x_ref, w_ref, b_ref, out_ref):
    out_ref[:] = (
        lax.dot_general(
            x_ref[:],
            w_ref[:],
            dimension_numbers=(((1,), (1,)), ((), ())),
            preferred_element_type=jnp.float32,
        )
        + b_ref[:]
    )


def _tc_matmul(x, W, b2d):
    return pl.pallas_call(
        _matmul_body,
        grid=(B // BS,),
        in_specs=[
            pl.BlockSpec((BS, L * D), lambda i: (i, 0)),
            pl.BlockSpec((V, L * D), lambda i: (0, 0)),
            pl.BlockSpec((1, V), lambda i: (0, 0)),
        ],
        out_specs=pl.BlockSpec((BS, V), lambda i: (i, 0)),
        out_shape=jax.ShapeDtypeStruct((B, V), jnp.float32),
    )(x, W, b2d)


def kernel(idx, emb, W, b):
    idx_chunks = idx.reshape(TOTAL_CHUNKS, CHUNK)
    x = _sc_gather(emb, idx_chunks)
    x = x.reshape(B, L * D)
    return _tc_matmul(x, W, b.reshape(1, V))


# speedup vs baseline: 1.7932x; 1.7932x over previous
import functools

import jax
import jax.numpy as jnp
from jax import lax
from jax.experimental import pallas as pl
from jax.experimental.pallas import tpu as pltpu
from jax.experimental.pallas import tpu_sc as plsc

B = 1024
L = 20
D = 128
V = 1000

N_IDX = B * L

_INFO = plsc.get_sparse_core_info()
NW = _INFO.num_cores * _INFO.num_subcores
ROWS_PER_W = N_IDX // NW
CHUNK = 128
N_CHUNKS = ROWS_PER_W // CHUNK
TOTAL_CHUNKS = N_IDX // CHUNK


def _gather_body(emb_hbm, idx_hbm, out_hbm, idx_v, rows_v, sem):
    wid = lax.axis_index("s") * _INFO.num_cores + lax.axis_index("c")
    base = wid * ROWS_PER_W
    pltpu.sync_copy(idx_hbm.at[pl.ds(base, ROWS_PER_W)], idx_v)
    copies = []
    for j in range(N_CHUNKS):
        copies.append(
            pltpu.async_copy(
                emb_hbm.at[idx_v.at[pl.ds(j * CHUNK, CHUNK)]],
                rows_v.at[pl.ds(j * CHUNK, CHUNK)],
                sem,
            )
        )
    for c in copies:
        c.wait()
    pltpu.sync_copy(rows_v, out_hbm.at[pl.ds(base, ROWS_PER_W)])


@functools.partial(
    pl.kernel,
    out_type=jax.ShapeDtypeStruct((N_IDX, D), jnp.float32),
    mesh=plsc.VectorSubcoreMesh(core_axis_name="c", subcore_axis_name="s"),
    scratch_types=[
        pltpu.VMEM((ROWS_PER_W,), jnp.int32),
        pltpu.VMEM((ROWS_PER_W, D), jnp.float32),
        pltpu.SemaphoreType.DMA,
    ],
)
def _sc_gather(emb_hbm, idx_hbm, out_hbm, idx_v, rows_v, sem):
    _gather_body(emb_hbm, idx_hbm, out_hbm, idx_v, rows_v, sem)


BS = 256


def _matmul_body(x_ref, w_ref, b_ref, out_ref):
    out_ref[:] = (
        lax.dot_general(
            x_ref[:],
            w_ref[:],
            dimension_numbers=(((1,), (1,)), ((), ())),
            preferred_element_type=jnp.float32,
        )
        + b_ref[:]
    )


def _tc_matmul(x, W, b2d):
    return pl.pallas_call(
        _matmul_body,
        grid=(B // BS,),
        in_specs=[
            pl.BlockSpec((BS, L * D), lambda i: (i, 0)),
            pl.BlockSpec((V, L * D), lambda i: (0, 0)),
            pl.BlockSpec((1, V), lambda i: (0, 0)),
        ],
        out_specs=pl.BlockSpec((BS, V), lambda i: (i, 0)),
        out_shape=jax.ShapeDtypeStruct((B, V), jnp.float32),
    )(x, W, b2d)


def kernel(idx, emb, W, b):
    x = _sc_gather(emb, idx.reshape(N_IDX))
    x = x.reshape(B, L * D)
    return _tc_matmul(x, W, b.reshape(1, V))
